# baseline (speedup 1.0000x reference)
import functools
import jax
import jax.numpy as jnp
from jax import lax
from jax.experimental import pallas as pl
from jax.experimental.pallas import tpu as pltpu
from jax.experimental.pallas import tpu_sc as plsc

B, S, D = 4, 4096, 1024
NC, NS = 2, 16
NW = NC * NS
SEQ_PER_W = S // NW
R = 16
NCHUNK = SEQ_PER_W // R
NXB = 5
NSTEP = NCHUNK * B


def _make():
    mesh = plsc.VectorSubcoreMesh(core_axis_name="c", subcore_axis_name="s")

    @functools.partial(
        pl.kernel,
        mesh=mesh,
        out_type=jax.ShapeDtypeStruct((B, S, D), jnp.float32),
        compiler_params=pltpu.CompilerParams(use_tc_tiling_on_sc=True),
        scratch_types=(
            [pltpu.VMEM((R, D), jnp.float32) for _ in range(2)]
            + [pltpu.VMEM((R, D), jnp.float32) for _ in range(NXB)]
            + [pltpu.SemaphoreType.DMA for _ in range(2 + 2 * NXB)]
        ),
    )
    def k(x_hbm, pe_hbm, out_hbm, pe0, pe1, x0, x1, x2, x3, x4,
          ps0, ps1, xs0, xs1, xs2, xs3, xs4, os0, os1, os2, os3, os4):
        pe_v = [pe0, pe1]
        x_v = [x0, x1, x2, x3, x4]
        pe_sem = [ps0, ps1]
        x_sem = [xs0, xs1, xs2, xs3, xs4]
        o_sem = [os0, os1, os2, os3, os4]

        wid = lax.axis_index("s") * NC + lax.axis_index("c")
        seq0 = wid * SEQ_PER_W

        def row0(c):
            return seq0 + c * R

        def issue_pe(c):
            return pltpu.async_copy(
                pe_hbm.at[pl.ds(row0(c), R), :], pe_v[c % 2], pe_sem[c % 2])

        def issue_x(j):
            c, b = j // B, j % B
            return pltpu.async_copy(
                x_hbm.at[b, pl.ds(row0(c), R), :], x_v[j % NXB], x_sem[j % NXB])

        def issue_out_half(j, h):
            c, b = j // B, j % B
            return pltpu.async_copy(
                x_v[j % NXB].at[pl.ds(h * (R // 2), R // 2), :],
                out_hbm.at[b, pl.ds(row0(c) + h * (R // 2), R // 2), :],
                o_sem[j % NXB])

        pe_h = [None] * NCHUNK
        x_h = [None] * NSTEP
        o_h = [None] * NSTEP

        pe_h[0] = issue_pe(0)
        pe_h[1] = issue_pe(1)
        x_h[0] = issue_x(0)
        x_h[1] = issue_x(1)
        x_h[2] = issue_x(2)

        for j in range(NSTEP):
            c, b = j // B, j % B
            xb = x_v[j % NXB]
            pb = pe_v[c % 2]
            if j + 3 < NSTEP:
                if j - 2 >= 0:
                    for hh in o_h[j - 2]:
                        hh.wait()
                x_h[j + 3] = issue_x(j + 3)
            if b == 0:
                pe_h[c].wait()
            x_h[j].wait()

            half_handles = []
            for h in range(2):
                @plsc.parallel_loop(h * (R // 2), (h + 1) * (R // 2), step=1)
                def add_row(r):
                    @plsc.parallel_loop(0, D, step=128)
                    def add_col(k16):
                        for u in range(8):
                            sl = pl.ds(k16 + u * 16, 16)
                            plsc.addupdate(xb.at[r, sl], pb[r, sl])

                half_handles.append(issue_out_half(j, h))
            o_h[j] = half_handles
            if b == B - 1 and c + 2 < NCHUNK:
                pe_h[c + 2] = issue_pe(c + 2)

        for j in range(NSTEP - 5, NSTEP):
            for hh in o_h[j]:
                hh.wait()

    return k


def kernel(x, pos_embedding):
    return _make()(x, pos_embedding)


# speedup vs baseline: 1.0294x; 1.0294x over previous
import functools
import jax
import jax.numpy as jnp
from jax import lax
from jax.experimental import pallas as pl
from jax.experimental.pallas import tpu as pltpu
from jax.experimental.pallas import tpu_sc as plsc

B, S, D = 4, 4096, 1024
NC, NS = 2, 16
NW = NC * NS
SEQ_PER_W = S // NW
R = 16
NCHUNK = SEQ_PER_W // R
NXB = 5
NSTEP = NCHUNK * B


def _make():
    mesh = plsc.VectorSubcoreMesh(core_axis_name="c", subcore_axis_name="s")

    @functools.partial(
        pl.kernel,
        mesh=mesh,
        out_type=jax.ShapeDtypeStruct((B, S, D), jnp.float32),
        compiler_params=pltpu.CompilerParams(use_tc_tiling_on_sc=True),
        scratch_types=(
            [pltpu.VMEM((R, D), jnp.float32) for _ in range(2)]
            + [pltpu.VMEM((R, D), jnp.float32) for _ in range(NXB)]
            + [pltpu.SemaphoreType.DMA for _ in range(2 + 2 * NXB)]
        ),
    )
    def k(x_hbm, pe_hbm, out_hbm, pe0, pe1, x0, x1, x2, x3, x4,
          ps0, ps1, xs0, xs1, xs2, xs3, xs4, os0, os1, os2, os3, os4):
        pe_v = [pe0, pe1]
        x_v = [x0, x1, x2, x3, x4]
        pe_sem = [ps0, ps1]
        x_sem = [xs0, xs1, xs2, xs3, xs4]
        o_sem = [os0, os1, os2, os3, os4]

        wid = lax.axis_index("s") * NC + lax.axis_index("c")
        seq0 = wid * SEQ_PER_W

        def row0(c):
            return seq0 + c * R

        def issue_pe(c):
            return pltpu.async_copy(
                pe_hbm.at[pl.ds(row0(c), R), :], pe_v[c % 2], pe_sem[c % 2])

        def issue_x(j):
            c, b = j // B, j % B
            return pltpu.async_copy(
                x_hbm.at[b, pl.ds(row0(c), R), :], x_v[j % NXB], x_sem[j % NXB])

        def issue_out(j):
            c, b = j // B, j % B
            return pltpu.async_copy(
                x_v[j % NXB], out_hbm.at[b, pl.ds(row0(c), R), :], o_sem[j % NXB])

        pe_h = [None] * NCHUNK
        x_h = [None] * NSTEP
        o_h = [None] * NSTEP

        pe_h[0] = issue_pe(0)
        pe_h[1] = issue_pe(1)
        x_h[0] = issue_x(0)
        x_h[1] = issue_x(1)
        x_h[2] = issue_x(2)

        for j in range(NSTEP):
            c, b = j // B, j % B
            xb = x_v[j % NXB]
            pb = pe_v[c % 2]
            if j + 3 < NSTEP:
                if j - 2 >= 0:
                    o_h[j - 2].wait()
                x_h[j + 3] = issue_x(j + 3)
            if b == 0:
                pe_h[c].wait()
            x_h[j].wait()

            @plsc.parallel_loop(0, R, step=1)
            def add_row(r):
                @plsc.parallel_loop(0, D, step=128)
                def add_col(k16):
                    for u in range(8):
                        sl = pl.ds(k16 + u * 16, 16)
                        plsc.addupdate(xb.at[r, sl], pb[r, sl])

            o_h[j] = issue_out(j)
            if b == B - 1 and c + 2 < NCHUNK:
                pe_h[c + 2] = issue_pe(c + 2)

        for j in range(NSTEP - 5, NSTEP):
            o_h[j].wait()

    return k


def kernel(x, pos_embedding):
    return _make()(x, pos_embedding)


# speedup vs baseline: 1.0357x; 1.0060x over previous
import functools
import jax
import jax.numpy as jnp
from jax import lax
from jax.experimental import pallas as pl
from jax.experimental.pallas import tpu as pltpu
from jax.experimental.pallas import tpu_sc as plsc

B, S, D = 4, 4096, 1024
NC, NS = 2, 16
NW = NC * NS
SEQ_PER_W = S // NW
R = 8
NCHUNK = SEQ_PER_W // R
NXB = 10
LEAD = 5
NSTEP = NCHUNK * B


def _make():
    mesh = plsc.VectorSubcoreMesh(core_axis_name="c", subcore_axis_name="s")

    @functools.partial(
        pl.kernel,
        mesh=mesh,
        out_type=jax.ShapeDtypeStruct((B, S, D), jnp.float32),
        compiler_params=pltpu.CompilerParams(use_tc_tiling_on_sc=True),
        scratch_types=(
            [pltpu.VMEM((R, D), jnp.float32) for _ in range(2 + NXB)]
            + [pltpu.SemaphoreType.DMA for _ in range(2 + 2 * NXB)]
        ),
    )
    def k(x_hbm, pe_hbm, out_hbm, *bufs):
        pe_v = list(bufs[0:2])
        x_v = list(bufs[2:2 + NXB])
        pe_sem = list(bufs[2 + NXB:4 + NXB])
        x_sem = list(bufs[4 + NXB:4 + 2 * NXB])
        o_sem = list(bufs[4 + 2 * NXB:4 + 3 * NXB])

        wid = lax.axis_index("s") * NC + lax.axis_index("c")
        seq0 = wid * SEQ_PER_W

        def row0(c):
            return seq0 + c * R

        def issue_pe(c):
            return pltpu.async_copy(
                pe_hbm.at[pl.ds(row0(c), R), :], pe_v[c % 2], pe_sem[c % 2])

        def issue_x(j):
            c, b = j // B, j % B
            return pltpu.async_copy(
                x_hbm.at[b, pl.ds(row0(c), R), :], x_v[j % NXB], x_sem[j % NXB])

        def issue_out(j):
            c, b = j // B, j % B
            return pltpu.async_copy(
                x_v[j % NXB], out_hbm.at[b, pl.ds(row0(c), R), :], o_sem[j % NXB])

        pe_h = [None] * NCHUNK
        x_h = [None] * NSTEP
        o_h = [None] * NSTEP

        pe_h[0] = issue_pe(0)
        pe_h[1] = issue_pe(1)
        for j in range(LEAD):
            x_h[j] = issue_x(j)

        for j in range(NSTEP):
            c, b = j // B, j % B
            xb = x_v[j % NXB]
            pb = pe_v[c % 2]
            if j + LEAD < NSTEP:
                if j + LEAD - NXB >= 0:
                    o_h[j + LEAD - NXB].wait()
                x_h[j + LEAD] = issue_x(j + LEAD)
            if b == 0:
                pe_h[c].wait()
            x_h[j].wait()

            @plsc.parallel_loop(0, R, step=1)
            def add_row(r):
                @plsc.parallel_loop(0, D, step=128)
                def add_col(k16):
                    for u in range(8):
                        sl = pl.ds(k16 + u * 16, 16)
                        plsc.addupdate(xb.at[r, sl], pb[r, sl])

            o_h[j] = issue_out(j)
            if b == B - 1 and c + 2 < NCHUNK:
                pe_h[c + 2] = issue_pe(c + 2)

        for j in range(NSTEP - NXB, NSTEP):
            o_h[j].wait()

    return k


def kernel(x, pos_embedding):
    return _make()(x, pos_embedding)
